# baseline (speedup 1.0000x reference)
import functools

import jax
import jax.numpy as jnp
from jax import lax
from jax.experimental import pallas as pl
from jax.experimental.pallas import tpu as pltpu
from jax.experimental.pallas import tpu_sc as plsc

D = 64
NBUF = 4
WSTRIDE = 136


def kernel(x, table, ln_weight, ln_bias):
    B, H = x.shape
    info = plsc.get_sparse_core_info()
    NC, NS = info.num_cores, info.num_subcores
    NW = NC * NS
    BB = B // NW
    assert B % NW == 0 and BB == 128 and H % NBUF == 0

    xt = x.T.astype(jnp.int32)

    mesh = plsc.VectorSubcoreMesh(core_axis_name="c", subcore_axis_name="s")

    @functools.partial(
        pl.kernel,
        mesh=mesh,
        out_type=jax.ShapeDtypeStruct((H, D, B), jnp.float32),
        compiler_params=pltpu.CompilerParams(
            needs_layout_passes=False, use_tc_tiling_on_sc=False
        ),
        scratch_types=(
            [pltpu.VMEM((H, BB), jnp.int32)]
            + [pltpu.VMEM((BB, D), jnp.float32) for _ in range(NBUF)]
            + [pltpu.VMEM((D, WSTRIDE), jnp.float32)]
            + [pltpu.VMEM((D, BB), jnp.float32) for _ in range(NBUF)]
            + [pltpu.VMEM((D,), jnp.float32) for _ in range(2)]
            + [pltpu.SemaphoreType.DMA for _ in range(2 * NBUF)]
        ),
    )
    def sc_kernel(xt_hbm, tbl_hbm, w_hbm, b_hbm, out_hbm, idx_v,
                  r0, r1, r2, r3, wscat, t0, t1, t2, t3, w_v, b_v,
                  g0, g1, g2, g3, o0, o1, o2, o3):
        rows_bufs = [r0, r1, r2, r3]
        wout_bufs = [t0, t1, t2, t3]
        gsem = [g0, g1, g2, g3]
        osem = [o0, o1, o2, o3]
        wid = lax.axis_index("s") * NC + lax.axis_index("c")
        col0 = wid * BB
        pltpu.sync_copy(xt_hbm.at[:, pl.ds(col0, BB)], idx_v)
        pltpu.sync_copy(w_hbm, w_v)
        pltpu.sync_copy(b_hbm, b_v)

        iot = lax.iota(jnp.int32, 16)
        perms = [(iot + sh) & 15 for sh in (8, 4, 2, 1)]
        wvecs = [w_v[pl.ds(16 * t, 16)] for t in range(4)]
        bvecs = [b_v[pl.ds(16 * t, 16)] for t in range(4)]
        drows = [iot + 16 * t for t in range(4)]

        def gstart(g, b):
            pltpu.async_copy(
                tbl_hbm.at[idx_v.at[g]],
                rows_bufs[b],
                gsem[b],
            )

        def gdrain(b):
            pltpu.make_async_copy(
                tbl_hbm.at[pl.ds(0, BB)], rows_bufs[b], gsem[b]
            ).wait()

        def odrain(b):
            pltpu.make_async_copy(
                out_hbm.at[0, :, pl.ds(0, BB)],
                wout_bufs[b],
                osem[b],
            ).wait()

        def compute(rows_ref, wout_ref):
            def row_body(r, c):
                v0 = rows_ref[r, pl.ds(0, 16)]
                v1 = rows_ref[r, pl.ds(16, 16)]
                v2 = rows_ref[r, pl.ds(32, 16)]
                v3 = rows_ref[r, pl.ds(48, 16)]
                s = (v0 + v1) + (v2 + v3)
                q = (v0 * v0 + v1 * v1) + (v2 * v2 + v3 * v3)
                for p in perms:
                    s = s + jnp.take_along_axis(s, p, axis=0,
                                                mode="promise_in_bounds")
                    q = q + jnp.take_along_axis(q, p, axis=0,
                                                mode="promise_in_bounds")
                mean = s * (1.0 / D)
                var = q * (1.0 / D) - mean * mean
                xv = var + 1e-5
                ii = plsc.bitcast(xv, jnp.int32)
                ii = 0x5F3759DF - (ii >> 1)
                y = plsc.bitcast(ii, jnp.float32)
                y = y * (1.5 - 0.5 * xv * y * y)
                y = y * (1.5 - 0.5 * xv * y * y)
                y = y * (1.5 - 0.5 * xv * y * y)
                u = mean * y
                col = jnp.broadcast_to(r, (16,)).astype(jnp.int32)
                for t, vt in enumerate((v0, v1, v2, v3)):
                    plsc.store_scatter(
                        wscat, [drows[t], col],
                        (vt * y - u) * wvecs[t] + bvecs[t],
                    )
                return c

            lax.fori_loop(0, BB, row_body, 0, unroll=4)

            def pack_body(d, c):
                for t in range(BB // 16):
                    wout_ref[d, pl.ds(16 * t, 16)] = \
                        wscat[d, pl.ds(16 * t, 16)]
                return c

            lax.fori_loop(0, D, pack_body, 0, unroll=4)

        gstart(0, 0)
        gstart(1, 1)

        def outer(i, carry):
            for b in range(NBUF):
                g = i * NBUF + b
                nb2 = (b + 2) % NBUF

                @pl.when(g + 2 < H)
                def _():
                    gstart(g + 2, nb2)

                gdrain(b)

                @pl.when(g >= NBUF)
                def _():
                    odrain(b)

                compute(rows_bufs[b], wout_bufs[b])
                pltpu.async_copy(
                    wout_bufs[b],
                    out_hbm.at[g, :, pl.ds(col0, BB)],
                    osem[b],
                )
            return carry

        lax.fori_loop(0, H // NBUF, outer, 0)
        for g in range(H - NBUF, H):
            odrain(g % NBUF)

    out_t = sc_kernel(xt, table, ln_weight, ln_bias)
    return jnp.transpose(out_t, (2, 0, 1))


# speedup vs baseline: 1.0019x; 1.0019x over previous
import functools

import jax
import jax.numpy as jnp
from jax import lax
from jax.experimental import pallas as pl
from jax.experimental.pallas import tpu as pltpu
from jax.experimental.pallas import tpu_sc as plsc

D = 64
NBUF = 4
WSTRIDE = 136


def kernel(x, table, ln_weight, ln_bias):
    B, H = x.shape
    info = plsc.get_sparse_core_info()
    NC, NS = info.num_cores, info.num_subcores
    NW = NC * NS
    BB = B // NW
    assert B % NW == 0 and BB == 128 and H % NBUF == 0

    xt = x.T.astype(jnp.int32)

    mesh = plsc.VectorSubcoreMesh(core_axis_name="c", subcore_axis_name="s")

    @functools.partial(
        pl.kernel,
        mesh=mesh,
        out_type=jax.ShapeDtypeStruct((H, D, B), jnp.float32),
        compiler_params=pltpu.CompilerParams(
            needs_layout_passes=False,
            use_tc_tiling_on_sc=False,
            disable_bounds_checks=True,
        ),
        scratch_types=(
            [pltpu.VMEM((H, BB), jnp.int32)]
            + [pltpu.VMEM((BB, D), jnp.float32) for _ in range(NBUF)]
            + [pltpu.VMEM((D, WSTRIDE), jnp.float32)]
            + [pltpu.VMEM((D, BB), jnp.float32) for _ in range(NBUF)]
            + [pltpu.VMEM((D,), jnp.float32) for _ in range(2)]
            + [pltpu.SemaphoreType.DMA for _ in range(2 * NBUF)]
        ),
    )
    def sc_kernel(xt_hbm, tbl_hbm, w_hbm, b_hbm, out_hbm, idx_v,
                  r0, r1, r2, r3, wscat, t0, t1, t2, t3, w_v, b_v,
                  g0, g1, g2, g3, o0, o1, o2, o3):
        rows_bufs = [r0, r1, r2, r3]
        wout_bufs = [t0, t1, t2, t3]
        gsem = [g0, g1, g2, g3]
        osem = [o0, o1, o2, o3]
        wid = lax.axis_index("s") * NC + lax.axis_index("c")
        col0 = wid * BB
        pltpu.sync_copy(xt_hbm.at[:, pl.ds(col0, BB)], idx_v)
        pltpu.sync_copy(w_hbm, w_v)
        pltpu.sync_copy(b_hbm, b_v)

        iot = lax.iota(jnp.int32, 16)
        perms = [(iot + sh) & 15 for sh in (8, 4, 2, 1)]
        wvecs = [w_v[pl.ds(16 * t, 16)] for t in range(4)]
        bvecs = [b_v[pl.ds(16 * t, 16)] for t in range(4)]
        drows = [iot + 16 * t for t in range(4)]

        def gstart(g, b):
            pltpu.async_copy(
                tbl_hbm.at[idx_v.at[g]],
                rows_bufs[b],
                gsem[b],
            )

        def gdrain(b):
            pltpu.make_async_copy(
                tbl_hbm.at[pl.ds(0, BB)], rows_bufs[b], gsem[b]
            ).wait()

        def odrain(b):
            pltpu.make_async_copy(
                out_hbm.at[0, :, pl.ds(0, BB)],
                wout_bufs[b],
                osem[b],
            ).wait()

        def compute(rows_ref, wout_ref):
            def row_body(r, c):
                v0 = rows_ref[r, pl.ds(0, 16)]
                v1 = rows_ref[r, pl.ds(16, 16)]
                v2 = rows_ref[r, pl.ds(32, 16)]
                v3 = rows_ref[r, pl.ds(48, 16)]
                s = (v0 + v1) + (v2 + v3)
                q = (v0 * v0 + v1 * v1) + (v2 * v2 + v3 * v3)
                for p in perms:
                    s = s + jnp.take_along_axis(s, p, axis=0,
                                                mode="promise_in_bounds")
                    q = q + jnp.take_along_axis(q, p, axis=0,
                                                mode="promise_in_bounds")
                mean = s * (1.0 / D)
                var = q * (1.0 / D) - mean * mean
                xv = var + 1e-5
                ii = plsc.bitcast(xv, jnp.int32)
                ii = 0x5F3759DF - (ii >> 1)
                y = plsc.bitcast(ii, jnp.float32)
                y = y * (1.5 - 0.5 * xv * y * y)
                y = y * (1.5 - 0.5 * xv * y * y)
                y = y * (1.5 - 0.5 * xv * y * y)
                u = mean * y
                col = jnp.broadcast_to(r, (16,)).astype(jnp.int32)
                for t, vt in enumerate((v0, v1, v2, v3)):
                    plsc.store_scatter(
                        wscat, [drows[t], col],
                        (vt * y - u) * wvecs[t] + bvecs[t],
                    )
                return c

            lax.fori_loop(0, BB, row_body, 0, unroll=4)

            def pack_body(d, c):
                for t in range(BB // 16):
                    wout_ref[d, pl.ds(16 * t, 16)] = \
                        wscat[d, pl.ds(16 * t, 16)]
                return c

            lax.fori_loop(0, D, pack_body, 0, unroll=4)

        gstart(0, 0)
        gstart(1, 1)

        def outer(i, carry):
            for b in range(NBUF):
                g = i * NBUF + b
                nb2 = (b + 2) % NBUF

                @pl.when(g + 2 < H)
                def _():
                    gstart(g + 2, nb2)

                gdrain(b)

                @pl.when(g >= NBUF)
                def _():
                    odrain(b)

                compute(rows_bufs[b], wout_bufs[b])
                pltpu.async_copy(
                    wout_bufs[b],
                    out_hbm.at[g, :, pl.ds(col0, BB)],
                    osem[b],
                )
            return carry

        lax.fori_loop(0, H // NBUF, outer, 0)
        for g in range(H - NBUF, H):
            odrain(g % NBUF)

    out_t = sc_kernel(xt, table, ln_weight, ln_bias)
    return jnp.transpose(out_t, (2, 0, 1))


# speedup vs baseline: 1.6364x; 1.6333x over previous
import functools

import jax
import jax.numpy as jnp
from jax import lax
from jax.experimental import pallas as pl
from jax.experimental.pallas import tpu as pltpu
from jax.experimental.pallas import tpu_sc as plsc

D = 64
SUB = 128
CHUNK = 256
NSUB = CHUNK // SUB
NBUF = 4


def kernel(x, table, ln_weight, ln_bias):
    B, H = x.shape
    nrow = B * H
    info = plsc.get_sparse_core_info()
    NC, NS = info.num_cores, info.num_subcores
    NW = NC * NS
    per_w = nrow // NW
    nchunks = per_w // CHUNK
    assert per_w % CHUNK == 0 and nrow % NW == 0 and nchunks % NBUF == 0

    x1 = x.reshape(nrow).astype(jnp.int32)

    mesh = plsc.VectorSubcoreMesh(core_axis_name="c", subcore_axis_name="s")

    @functools.partial(
        pl.kernel,
        mesh=mesh,
        out_type=jax.ShapeDtypeStruct((nrow, D), jnp.float32),
        compiler_params=pltpu.CompilerParams(
            needs_layout_passes=False,
            use_tc_tiling_on_sc=False,
            disable_bounds_checks=True,
        ),
        scratch_types=(
            [pltpu.VMEM((per_w,), jnp.int32)]
            + [pltpu.VMEM((CHUNK, D), jnp.float32) for _ in range(NBUF)]
            + [pltpu.VMEM((D,), jnp.float32) for _ in range(2)]
            + [pltpu.SemaphoreType.DMA for _ in range(2 * NBUF)]
        ),
    )
    def sc_kernel(x_hbm, tbl_hbm, w_hbm, b_hbm, out_hbm, idx_v,
                  r0, r1, r2, r3, w_v, b_v,
                  g0, g1, g2, g3, o0, o1, o2, o3):
        rows_bufs = [r0, r1, r2, r3]
        gsem = [g0, g1, g2, g3]
        osem = [o0, o1, o2, o3]
        wid = lax.axis_index("s") * NC + lax.axis_index("c")
        row0 = wid * per_w
        pltpu.sync_copy(x_hbm.at[pl.ds(row0, per_w)], idx_v)
        pltpu.sync_copy(w_hbm, w_v)
        pltpu.sync_copy(b_hbm, b_v)

        iot = lax.iota(jnp.int32, 16)
        perms = [(iot + sh) & 15 for sh in (8, 4, 2, 1)]
        wvecs = [w_v[pl.ds(16 * t, 16)] for t in range(4)]
        bvecs = [b_v[pl.ds(16 * t, 16)] for t in range(4)]

        def gstart(g, b):
            for j in range(NSUB):
                pltpu.async_copy(
                    tbl_hbm.at[idx_v.at[pl.ds(g * CHUNK + j * SUB, SUB)]],
                    rows_bufs[b].at[pl.ds(j * SUB, SUB)],
                    gsem[b],
                )

        def drain(sem, b):
            pltpu.make_async_copy(
                tbl_hbm.at[pl.ds(0, CHUNK)], rows_bufs[b], sem
            ).wait()

        def compute(rows_ref):
            def row_body(r, c):
                v0 = rows_ref[r, pl.ds(0, 16)]
                v1 = rows_ref[r, pl.ds(16, 16)]
                v2 = rows_ref[r, pl.ds(32, 16)]
                v3 = rows_ref[r, pl.ds(48, 16)]
                s = (v0 + v1) + (v2 + v3)
                q = (v0 * v0 + v1 * v1) + (v2 * v2 + v3 * v3)
                for p in perms:
                    s = s + jnp.take_along_axis(s, p, axis=0,
                                                mode="promise_in_bounds")
                    q = q + jnp.take_along_axis(q, p, axis=0,
                                                mode="promise_in_bounds")
                mean = s * (1.0 / D)
                var = q * (1.0 / D) - mean * mean
                xv = var + 1e-5
                ii = plsc.bitcast(xv, jnp.int32)
                ii = 0x5F3759DF - (ii >> 1)
                y = plsc.bitcast(ii, jnp.float32)
                y = y * (1.5 - 0.5 * xv * y * y)
                y = y * (1.5 - 0.5 * xv * y * y)
                y = y * (1.5 - 0.5 * xv * y * y)
                u = mean * y
                for t, vt in enumerate((v0, v1, v2, v3)):
                    rows_ref[r, pl.ds(16 * t, 16)] = \
                        (vt * y - u) * wvecs[t] + bvecs[t]
                return c

            lax.fori_loop(0, CHUNK, row_body, 0, unroll=8)

        gstart(0, 0)
        gstart(1, 1)

        def outer(i, carry):
            for b in range(NBUF):
                g = i * NBUF + b
                nb2 = (b + 2) % NBUF

                @pl.when(g >= 2)
                def _():
                    drain(osem[nb2], nb2)

                @pl.when(g + 2 < nchunks)
                def _():
                    gstart(g + 2, nb2)

                drain(gsem[b], b)
                compute(rows_bufs[b])
                pltpu.async_copy(
                    rows_bufs[b],
                    out_hbm.at[pl.ds(row0 + g * CHUNK, CHUNK)],
                    osem[b],
                )
            return carry

        lax.fori_loop(0, nchunks // NBUF, outer, 0)
        for g in range(nchunks - 2, nchunks):
            drain(osem[g % NBUF], g % NBUF)

    out = sc_kernel(x1, table, ln_weight, ln_bias)
    return out.reshape(B, H, D)
